# baseline (speedup 1.0000x reference)
import functools

import jax
import jax.numpy as jnp
import numpy as np
from jax import lax
from jax.experimental import pallas as pl
from jax.experimental.pallas import tpu as pltpu
from jax.experimental.pallas import tpu_sc as plsc

_K = 8
_N = 64
_LANES = 16
_NUM_CORES = 2
_NUM_SUBCORES = 16
_NUM_WORKERS = _NUM_CORES * _NUM_SUBCORES
_BR = _K + 1


def _sc_body(scores_hbm, u_hbm, mask_hbm, marg_hbm, raw_v, th_v, u_v, w_v,
             btab, mask_v, marg_v, *, groups_per_worker):
    wid = lax.axis_index("s") * _NUM_CORES + lax.axis_index("c")
    lane = jnp.arange(_LANES, dtype=jnp.int32)
    tidx = (lane >> 1) * (2 * _N) + (lane & 1)
    one = jnp.ones((_LANES,), jnp.float32)
    zero = jnp.zeros((_LANES,), jnp.float32)

    def do_group(i, carry):
        g = wid * groups_per_worker + i
        pltpu.sync_copy(scores_hbm.at[g], raw_v)
        pltpu.sync_copy(u_hbm.at[g], u_v)

        acc = plsc.load_gather(raw_v, [tidx])
        th_v[0] = acc
        for c in range(1, _N):
            th = plsc.load_gather(raw_v, [tidx + 2 * c])
            th_v[c] = th
            acc = acc + th
        mu = acc * jnp.float32(1.0 / _N)

        bs = [one] + [zero] * _K
        for r in range(_BR):
            btab[_N * _BR + r] = bs[r]
        for j in range(_N - 1, -1, -1):
            wj = jnp.exp(th_v[j] - mu)
            w_v[j] = wj
            for r in range(_K, 0, -1):
                bs[r] = bs[r] + wj * bs[r - 1]
            for r in range(_BR):
                btab[j * _BR + r] = bs[r]

        inv_ek = one / btab[_K]

        fs = [one] + [zero] * (_K - 1)
        rem = jnp.full((_LANES,), _K, jnp.int32)
        for j in range(_N):
            wj = w_v[j]
            base1 = (j + 1) * _BR
            dot = fs[_K - 1] + fs[0] * btab[base1 + _K - 1]
            for r in range(1, _K - 1):
                dot = dot + fs[r] * btab[base1 + _K - 1 - r]
            pm = wj * dot * inv_ek
            for r in range(_K - 1, 0, -1):
                fs[r] = fs[r] + wj * fs[r - 1]

            jv = jnp.full((_LANES,), j * _BR, jnp.int32)
            b_cur = plsc.load_gather(btab, [jv + rem, lane])
            b_inc = plsc.load_gather(
                btab, [jv + (_BR + jnp.maximum(rem - 1, 0)), lane])
            p = wj * b_inc / jnp.maximum(b_cur, jnp.float32(1e-35))
            p = jnp.minimum(p, jnp.float32(1.0))
            p = jnp.where(rem > 0, p, jnp.float32(0.0))
            inc = u_v[j] < p
            rem = rem - inc.astype(jnp.int32)
            oidx = tidx + 2 * j
            plsc.store_scatter(mask_v, [oidx], inc.astype(jnp.float32))
            plsc.store_scatter(marg_v, [oidx], pm)

        pltpu.sync_copy(mask_v, mask_hbm.at[g])
        pltpu.sync_copy(marg_v, marg_hbm.at[g])
        return carry

    lax.fori_loop(0, groups_per_worker, do_group, 0)


@jax.jit
def _sc_sampler(scores2, u3):
    g_total = scores2.shape[0]
    chunk = scores2.shape[1]
    groups_per_worker = g_total // _NUM_WORKERS
    mesh = plsc.VectorSubcoreMesh(
        core_axis_name="c", subcore_axis_name="s",
        num_cores=_NUM_CORES, num_subcores=_NUM_SUBCORES)
    body = functools.partial(_sc_body, groups_per_worker=groups_per_worker)
    f = pl.kernel(
        body,
        out_type=(
            jax.ShapeDtypeStruct((g_total, chunk), jnp.float32),
            jax.ShapeDtypeStruct((g_total, chunk), jnp.float32),
        ),
        mesh=mesh,
        compiler_params=pltpu.CompilerParams(needs_layout_passes=False),
        scratch_types=[
            pltpu.VMEM((chunk,), jnp.float32),
            pltpu.VMEM((_N, _LANES), jnp.float32),
            pltpu.VMEM((_N, _LANES), jnp.float32),
            pltpu.VMEM((_N, _LANES), jnp.float32),
            pltpu.VMEM(((_N + 1) * _BR, _LANES), jnp.float32),
            pltpu.VMEM((chunk,), jnp.float32),
            pltpu.VMEM((chunk,), jnp.float32),
        ],
    )
    return f(scores2, u3)


_U3_CACHE = {}


def _rotl32(x, r):
    return ((x << np.uint32(r)) | (x >> np.uint32(32 - r))).astype(np.uint32)


def _threefry2x32(k1, k2, x0, x1):
    ks = [np.uint32(k1), np.uint32(k2),
          np.uint32(np.uint32(k1) ^ np.uint32(k2) ^ np.uint32(0x1BD11BDA))]
    rots = [[13, 15, 26, 6], [17, 29, 16, 24]]
    x0 = (x0 + ks[0]).astype(np.uint32)
    x1 = (x1 + ks[1]).astype(np.uint32)
    for i in range(5):
        for r in rots[i % 2]:
            x0 = (x0 + x1).astype(np.uint32)
            x1 = _rotl32(x1, r)
            x1 = (x1 ^ x0).astype(np.uint32)
        x0 = (x0 + ks[(i + 1) % 3]).astype(np.uint32)
        x1 = (x1 + ks[(i + 2) % 3] + np.uint32(i + 1)).astype(np.uint32)
    return x0, x1


def _np_uniform_bits(seed, size):
    k1, k2 = np.uint32(seed >> 32), np.uint32(seed & 0xFFFFFFFF)
    if jax.config.jax_threefry_partitionable:
        cnt = np.arange(size, dtype=np.uint64)
        x0, x1 = _threefry2x32(k1, k2,
                               (cnt >> np.uint64(32)).astype(np.uint32),
                               (cnt & np.uint64(0xFFFFFFFF)).astype(np.uint32))
        bits = (x0 ^ x1).astype(np.uint32)
    else:
        cnt = np.arange(size, dtype=np.uint32)
        x0, x1 = _threefry2x32(k1, k2, cnt[:size // 2], cnt[size // 2:])
        bits = np.concatenate([x0, x1])
    fl = ((bits >> np.uint32(9)) | np.uint32(0x3F800000)).view(np.float32)
    return np.maximum(np.float32(0.0), fl - np.float32(1.0))


def _uniform_blocks(choices, rows):
    key = (choices, rows, bool(jax.config.jax_threefry_partitionable))
    if key not in _U3_CACHE:
        uni = _np_uniform_bits(42, choices * rows)
        groups = rows // _LANES
        u3 = uni.reshape(choices, groups, _LANES).transpose(1, 0, 2)
        _U3_CACHE[key] = np.ascontiguousarray(u3)
    return _U3_CACHE[key]


def kernel(scores):
    nnodes, choices, ensemble = scores.shape
    rows = nnodes * ensemble
    groups = rows // _LANES
    scores2 = scores.reshape(groups, _LANES // ensemble * choices * ensemble)
    u3 = jnp.asarray(_uniform_blocks(choices, rows))

    mask2, marg2 = _sc_sampler(scores2, u3)

    new_mask = mask2.reshape(1, nnodes, choices, ensemble)
    new_marg = marg2.reshape(nnodes, choices, ensemble)
    return new_mask, new_marg


# speedup vs baseline: 1.0223x; 1.0223x over previous
import functools

import jax
import jax.numpy as jnp
import numpy as np
from jax import lax
from jax.experimental import pallas as pl
from jax.experimental.pallas import tpu as pltpu
from jax.experimental.pallas import tpu_sc as plsc

_K = 8
_N = 64
_LANES = 16
_NUM_CORES = 2
_NUM_SUBCORES = 16
_NUM_WORKERS = _NUM_CORES * _NUM_SUBCORES
_BR = _K + 1
_CHUNK = 1024


def _sc_body(scores_hbm, u_hbm, mask_hbm, marg_hbm, raw_v, th_v, u_v, w_v,
             btab, mask_v, marg_v, *, groups_per_worker):
    wid = lax.axis_index("s") * _NUM_CORES + lax.axis_index("c")
    lane = jnp.arange(_LANES, dtype=jnp.int32)
    tidx = (lane >> 1) * (2 * _N) + (lane & 1)
    one = jnp.ones((_LANES,), jnp.float32)
    zero = jnp.zeros((_LANES,), jnp.float32)

    def do_group(i, carry):
        g = wid * groups_per_worker + i
        pltpu.sync_copy(scores_hbm.at[pl.ds(g * _CHUNK, _CHUNK)], raw_v)
        pltpu.sync_copy(u_hbm.at[pl.ds(g * _CHUNK, _CHUNK)], u_v)

        acc = plsc.load_gather(raw_v, [tidx])
        th_v[0] = acc
        for c in range(1, _N):
            th = plsc.load_gather(raw_v, [tidx + 2 * c])
            th_v[c] = th
            acc = acc + th
        mu = acc * jnp.float32(1.0 / _N)

        bs = [one] + [zero] * _K
        for r in range(_BR):
            btab[_N * _BR + r] = bs[r]
        for j in range(_N - 1, -1, -1):
            wj = jnp.exp(th_v[j] - mu)
            w_v[j] = wj
            for r in range(_K, 0, -1):
                bs[r] = bs[r] + wj * bs[r - 1]
            for r in range(_BR):
                btab[j * _BR + r] = bs[r]

        inv_ek = one / btab[_K]

        fs = [one] + [zero] * (_K - 1)
        rem = jnp.full((_LANES,), _K, jnp.int32)
        for j in range(_N):
            wj = w_v[j]
            base1 = (j + 1) * _BR
            dot = fs[_K - 1] + fs[0] * btab[base1 + _K - 1]
            for r in range(1, _K - 1):
                dot = dot + fs[r] * btab[base1 + _K - 1 - r]
            pm = wj * dot * inv_ek
            for r in range(_K - 1, 0, -1):
                fs[r] = fs[r] + wj * fs[r - 1]

            jv = jnp.full((_LANES,), j * _BR, jnp.int32)
            b_cur = plsc.load_gather(btab, [jv + rem, lane])
            b_inc = plsc.load_gather(
                btab, [jv + (_BR + jnp.maximum(rem - 1, 0)), lane])
            p = wj * b_inc / jnp.maximum(b_cur, jnp.float32(1e-35))
            p = jnp.minimum(p, jnp.float32(1.0))
            p = jnp.where(rem > 0, p, jnp.float32(0.0))
            inc = u_v[pl.ds(j * _LANES, _LANES)] < p
            rem = rem - inc.astype(jnp.int32)
            oidx = tidx + 2 * j
            plsc.store_scatter(mask_v, [oidx], inc.astype(jnp.float32))
            plsc.store_scatter(marg_v, [oidx], pm)

        pltpu.sync_copy(mask_v, mask_hbm.at[pl.ds(g * _CHUNK, _CHUNK)])
        pltpu.sync_copy(marg_v, marg_hbm.at[pl.ds(g * _CHUNK, _CHUNK)])
        return carry

    lax.fori_loop(0, groups_per_worker, do_group, 0)


@jax.jit
def _sc_sampler(scores1, u1):
    g_total = scores1.shape[0] // _CHUNK
    groups_per_worker = g_total // _NUM_WORKERS
    mesh = plsc.VectorSubcoreMesh(
        core_axis_name="c", subcore_axis_name="s",
        num_cores=_NUM_CORES, num_subcores=_NUM_SUBCORES)
    body = functools.partial(_sc_body, groups_per_worker=groups_per_worker)
    f = pl.kernel(
        body,
        out_type=(
            jax.ShapeDtypeStruct((g_total * _CHUNK,), jnp.float32),
            jax.ShapeDtypeStruct((g_total * _CHUNK,), jnp.float32),
        ),
        mesh=mesh,
        compiler_params=pltpu.CompilerParams(needs_layout_passes=False),
        scratch_types=[
            pltpu.VMEM((_CHUNK,), jnp.float32),
            pltpu.VMEM((_N, _LANES), jnp.float32),
            pltpu.VMEM((_CHUNK,), jnp.float32),
            pltpu.VMEM((_N, _LANES), jnp.float32),
            pltpu.VMEM(((_N + 1) * _BR, _LANES), jnp.float32),
            pltpu.VMEM((_CHUNK,), jnp.float32),
            pltpu.VMEM((_CHUNK,), jnp.float32),
        ],
    )
    return f(scores1, u1)


_U3_CACHE = {}


def _rotl32(x, r):
    return ((x << np.uint32(r)) | (x >> np.uint32(32 - r))).astype(np.uint32)


def _threefry2x32(k1, k2, x0, x1):
    ks = [np.uint32(k1), np.uint32(k2),
          np.uint32(np.uint32(k1) ^ np.uint32(k2) ^ np.uint32(0x1BD11BDA))]
    rots = [[13, 15, 26, 6], [17, 29, 16, 24]]
    x0 = (x0 + ks[0]).astype(np.uint32)
    x1 = (x1 + ks[1]).astype(np.uint32)
    for i in range(5):
        for r in rots[i % 2]:
            x0 = (x0 + x1).astype(np.uint32)
            x1 = _rotl32(x1, r)
            x1 = (x1 ^ x0).astype(np.uint32)
        x0 = (x0 + ks[(i + 1) % 3]).astype(np.uint32)
        x1 = (x1 + ks[(i + 2) % 3] + np.uint32(i + 1)).astype(np.uint32)
    return x0, x1


def _np_uniform_bits(seed, size):
    k1, k2 = np.uint32(seed >> 32), np.uint32(seed & 0xFFFFFFFF)
    if jax.config.jax_threefry_partitionable:
        cnt = np.arange(size, dtype=np.uint64)
        x0, x1 = _threefry2x32(k1, k2,
                               (cnt >> np.uint64(32)).astype(np.uint32),
                               (cnt & np.uint64(0xFFFFFFFF)).astype(np.uint32))
        bits = (x0 ^ x1).astype(np.uint32)
    else:
        cnt = np.arange(size, dtype=np.uint32)
        x0, x1 = _threefry2x32(k1, k2, cnt[:size // 2], cnt[size // 2:])
        bits = np.concatenate([x0, x1])
    fl = ((bits >> np.uint32(9)) | np.uint32(0x3F800000)).view(np.float32)
    return np.maximum(np.float32(0.0), fl - np.float32(1.0))


def _uniform_blocks(choices, rows):
    key = (choices, rows, bool(jax.config.jax_threefry_partitionable))
    if key not in _U3_CACHE:
        uni = _np_uniform_bits(42, choices * rows)
        groups = rows // _LANES
        u3 = uni.reshape(choices, groups, _LANES).transpose(1, 0, 2)
        _U3_CACHE[key] = np.ascontiguousarray(u3)
    return _U3_CACHE[key]


def kernel(scores):
    nnodes, choices, ensemble = scores.shape
    rows = nnodes * ensemble
    scores1 = scores.reshape(rows * choices)
    u1 = jnp.asarray(_uniform_blocks(choices, rows).reshape(-1))

    mask1, marg1 = _sc_sampler(scores1, u1)

    new_mask = mask1.reshape(1, nnodes, choices, ensemble)
    new_marg = marg1.reshape(nnodes, choices, ensemble)
    return new_mask, new_marg


# speedup vs baseline: 5.3061x; 5.1904x over previous
import functools

import jax
import jax.numpy as jnp
import numpy as np
from jax import lax
from jax.experimental import pallas as pl
from jax.experimental.pallas import tpu as pltpu
from jax.experimental.pallas import tpu_sc as plsc

_K = 8
_N = 64
_LANES = 16
_NUM_CORES = 2
_NUM_SUBCORES = 16
_NUM_WORKERS = _NUM_CORES * _NUM_SUBCORES
_BR = _K + 1


def _sc_body(theta_hbm, u_hbm, mask_hbm, marg_hbm, th_v, u_v, w_v, btab,
             mask_v, marg_v, *, groups_per_worker):
    wid = lax.axis_index("s") * _NUM_CORES + lax.axis_index("c")
    lane = jnp.arange(_LANES, dtype=jnp.int32)
    one = jnp.ones((_LANES,), jnp.float32)
    zero = jnp.zeros((_LANES,), jnp.float32)

    def do_group(i, carry):
        g = wid * groups_per_worker + i
        pltpu.sync_copy(theta_hbm.at[g], th_v)
        pltpu.sync_copy(u_hbm.at[g], u_v)

        acc = th_v[0]
        for j in range(1, _N):
            acc = acc + th_v[j]
        mu = acc * jnp.float32(1.0 / _N)

        bs = [one] + [zero] * _K
        for r in range(_BR):
            btab[_N * _BR + r] = bs[r]
        for j in range(_N - 1, -1, -1):
            wj = jnp.exp(th_v[j] - mu)
            w_v[j] = wj
            for r in range(_K, 0, -1):
                bs[r] = bs[r] + wj * bs[r - 1]
            for r in range(_BR):
                btab[j * _BR + r] = bs[r]

        inv_ek = one / btab[_K]

        fs = [one] + [zero] * (_K - 1)
        rem = jnp.full((_LANES,), _K, jnp.int32)
        for j in range(_N):
            wj = w_v[j]
            base1 = (j + 1) * _BR
            dot = fs[_K - 1] + fs[0] * btab[base1 + _K - 1]
            for r in range(1, _K - 1):
                dot = dot + fs[r] * btab[base1 + _K - 1 - r]
            marg_v[j] = wj * dot * inv_ek
            for r in range(_K - 1, 0, -1):
                fs[r] = fs[r] + wj * fs[r - 1]

            jv = jnp.full((_LANES,), j * _BR, jnp.int32)
            b_cur = plsc.load_gather(btab, [jv + rem, lane])
            b_inc = plsc.load_gather(
                btab, [jv + (_BR + jnp.maximum(rem - 1, 0)), lane])
            p = wj * b_inc / jnp.maximum(b_cur, jnp.float32(1e-35))
            p = jnp.minimum(p, jnp.float32(1.0))
            p = jnp.where(rem > 0, p, jnp.float32(0.0))
            inc = u_v[j] < p
            rem = rem - inc.astype(jnp.int32)
            mask_v[j] = inc.astype(jnp.float32)

        pltpu.sync_copy(mask_v, mask_hbm.at[g])
        pltpu.sync_copy(marg_v, marg_hbm.at[g])
        return carry

    lax.fori_loop(0, groups_per_worker, do_group, 0)


@jax.jit
def _sc_sampler(theta3, u3):
    g_total = theta3.shape[0]
    groups_per_worker = g_total // _NUM_WORKERS
    mesh = plsc.VectorSubcoreMesh(
        core_axis_name="c", subcore_axis_name="s",
        num_cores=_NUM_CORES, num_subcores=_NUM_SUBCORES)
    body = functools.partial(_sc_body, groups_per_worker=groups_per_worker)
    f = pl.kernel(
        body,
        out_type=(
            jax.ShapeDtypeStruct((g_total, _N, _LANES), jnp.float32),
            jax.ShapeDtypeStruct((g_total, _N, _LANES), jnp.float32),
        ),
        mesh=mesh,
        compiler_params=pltpu.CompilerParams(needs_layout_passes=False),
        scratch_types=[
            pltpu.VMEM((_N, _LANES), jnp.float32),
            pltpu.VMEM((_N, _LANES), jnp.float32),
            pltpu.VMEM((_N, _LANES), jnp.float32),
            pltpu.VMEM(((_N + 1) * _BR, _LANES), jnp.float32),
            pltpu.VMEM((_N, _LANES), jnp.float32),
            pltpu.VMEM((_N, _LANES), jnp.float32),
        ],
    )
    return f(theta3, u3)


_U3_CACHE = {}


def _rotl32(x, r):
    return ((x << np.uint32(r)) | (x >> np.uint32(32 - r))).astype(np.uint32)


def _threefry2x32(k1, k2, x0, x1):
    ks = [np.uint32(k1), np.uint32(k2),
          np.uint32(np.uint32(k1) ^ np.uint32(k2) ^ np.uint32(0x1BD11BDA))]
    rots = [[13, 15, 26, 6], [17, 29, 16, 24]]
    x0 = (x0 + ks[0]).astype(np.uint32)
    x1 = (x1 + ks[1]).astype(np.uint32)
    for i in range(5):
        for r in rots[i % 2]:
            x0 = (x0 + x1).astype(np.uint32)
            x1 = _rotl32(x1, r)
            x1 = (x1 ^ x0).astype(np.uint32)
        x0 = (x0 + ks[(i + 1) % 3]).astype(np.uint32)
        x1 = (x1 + ks[(i + 2) % 3] + np.uint32(i + 1)).astype(np.uint32)
    return x0, x1


def _np_uniform_bits(seed, size):
    k1, k2 = np.uint32(seed >> 32), np.uint32(seed & 0xFFFFFFFF)
    if jax.config.jax_threefry_partitionable:
        cnt = np.arange(size, dtype=np.uint64)
        x0, x1 = _threefry2x32(k1, k2,
                               (cnt >> np.uint64(32)).astype(np.uint32),
                               (cnt & np.uint64(0xFFFFFFFF)).astype(np.uint32))
        bits = (x0 ^ x1).astype(np.uint32)
    else:
        cnt = np.arange(size, dtype=np.uint32)
        x0, x1 = _threefry2x32(k1, k2, cnt[:size // 2], cnt[size // 2:])
        bits = np.concatenate([x0, x1])
    fl = ((bits >> np.uint32(9)) | np.uint32(0x3F800000)).view(np.float32)
    return np.maximum(np.float32(0.0), fl - np.float32(1.0))


def _uniform_blocks(choices, rows):
    key = (choices, rows, bool(jax.config.jax_threefry_partitionable))
    if key not in _U3_CACHE:
        uni = _np_uniform_bits(42, choices * rows)
        groups = rows // _LANES
        u3 = uni.reshape(choices, groups, _LANES).transpose(1, 0, 2)
        _U3_CACHE[key] = np.ascontiguousarray(u3)
    return _U3_CACHE[key]


def kernel(scores):
    nnodes, choices, ensemble = scores.shape
    rows = nnodes * ensemble
    groups = rows // _LANES
    flat = jnp.transpose(scores, (0, 2, 1)).reshape(rows, choices)
    theta3 = flat.reshape(groups, _LANES, choices).transpose(0, 2, 1)
    u3 = jnp.asarray(_uniform_blocks(choices, rows))

    mask3, marg3 = _sc_sampler(theta3, u3)

    samples = mask3.transpose(0, 2, 1).reshape(rows, choices)
    marg = marg3.transpose(0, 2, 1).reshape(rows, choices)
    new_mask = samples.reshape(1, nnodes, ensemble, choices)
    new_mask = jnp.transpose(new_mask, (0, 1, 3, 2))
    new_marg = jnp.transpose(marg.reshape(nnodes, ensemble, choices),
                             (0, 2, 1))
    return new_mask, new_marg


# speedup vs baseline: 5.8867x; 1.1094x over previous
import functools

import jax
import jax.numpy as jnp
import numpy as np
from jax import lax
from jax.experimental import pallas as pl
from jax.experimental.pallas import tpu as pltpu
from jax.experimental.pallas import tpu_sc as plsc

_K = 8
_N = 64
_LANES = 16
_NUM_CORES = 2
_NUM_SUBCORES = 16
_NUM_WORKERS = _NUM_CORES * _NUM_SUBCORES
_BR = _K + 1
_PAIR = 1


def _sc_body(theta_hbm, u_hbm, mask_hbm, marg_hbm,
             th_v0, u_v0, w_v0, btab0, mask_v0, marg_v0,
             in_sem, *, groups_per_worker):
    wid = lax.axis_index("s") * _NUM_CORES + lax.axis_index("c")
    lane = jnp.arange(_LANES, dtype=jnp.int32)
    one = jnp.ones((_LANES,), jnp.float32)
    zero = jnp.zeros((_LANES,), jnp.float32)
    A = dict(th=th_v0, u=u_v0, w=w_v0, bt=btab0, mk=mask_v0, mg=marg_v0)
    pair = (A,)

    for s in pair:
        for j in range(_N + 1):
            s["bt"][j * _BR] = one
        for r in range(1, _BR):
            s["bt"][_N * _BR + r] = zero

    def do_pair(i, carry):
        g0 = wid * groups_per_worker + _PAIR * i
        copies = []
        for t, s in enumerate(pair):
            copies.append(pltpu.async_copy(theta_hbm.at[g0 + t], s["th"],
                                           in_sem))
            copies.append(pltpu.async_copy(u_hbm.at[g0 + t], s["u"], in_sem))
        for c in copies:
            c.wait()

        acc = [s["th"][0] for s in pair]
        for j in range(1, _N):
            acc = [a + s["th"][j] for a, s in zip(acc, pair)]
        mu = [a * jnp.float32(1.0 / _N) for a in acc]

        bs = [[one] + [zero] * _K for _ in pair]
        for j in range(_N - 1, -1, -1):
            wj = [jnp.exp(s["th"][j] - m) for s, m in zip(pair, mu)]
            for t, s in enumerate(pair):
                s["w"][j] = wj[t]
                for r in range(_K, 0, -1):
                    bs[t][r] = bs[t][r] + wj[t] * bs[t][r - 1]
                for r in range(1, _BR):
                    s["bt"][j * _BR + r] = bs[t][r]

        inv_ek = [one / s["bt"][_K] for s in pair]

        fs = [[one] + [zero] * (_K - 1) for _ in pair]
        rem = [jnp.full((_LANES,), _K, jnp.int32) for _ in pair]
        for j in range(_N):
            base1 = (j + 1) * _BR
            jv = jnp.full((_LANES,), j * _BR, jnp.int32)
            for t, s in enumerate(pair):
                wj = s["w"][j]
                dot = fs[t][_K - 1] + fs[t][0] * s["bt"][base1 + _K - 1]
                for r in range(1, _K - 1):
                    dot = dot + fs[t][r] * s["bt"][base1 + _K - 1 - r]
                s["mg"][j] = wj * dot * inv_ek[t]
                for r in range(_K - 1, 0, -1):
                    fs[t][r] = fs[t][r] + wj * fs[t][r - 1]

                idx = jv + rem[t]
                b_cur = plsc.load_gather(s["bt"], [idx, lane])
                b_inc = plsc.load_gather(s["bt"], [idx + _K, lane])
                p = wj * b_inc / jnp.maximum(b_cur, jnp.float32(1e-35))
                inc = (s["u"][j] < p) & (rem[t] > 0)
                rem[t] = rem[t] - inc.astype(jnp.int32)
                s["mk"][j] = inc.astype(jnp.float32)

        for t, s in enumerate(pair):
            pltpu.sync_copy(s["mk"], mask_hbm.at[g0 + t])
            pltpu.sync_copy(s["mg"], marg_hbm.at[g0 + t])
        return carry

    lax.fori_loop(0, groups_per_worker // _PAIR, do_pair, 0)


@jax.jit
def _sc_sampler(theta3, u3):
    g_total = theta3.shape[0]
    groups_per_worker = g_total // _NUM_WORKERS
    mesh = plsc.VectorSubcoreMesh(
        core_axis_name="c", subcore_axis_name="s",
        num_cores=_NUM_CORES, num_subcores=_NUM_SUBCORES)
    body = functools.partial(_sc_body, groups_per_worker=groups_per_worker)
    group_bufs = [
        pltpu.VMEM((_N, _LANES), jnp.float32),
        pltpu.VMEM((_N, _LANES), jnp.float32),
        pltpu.VMEM((_N, _LANES), jnp.float32),
        pltpu.VMEM(((_N + 1) * _BR, _LANES), jnp.float32),
        pltpu.VMEM((_N, _LANES), jnp.float32),
        pltpu.VMEM((_N, _LANES), jnp.float32),
    ]
    f = pl.kernel(
        body,
        out_type=(
            jax.ShapeDtypeStruct((g_total, _N, _LANES), jnp.float32),
            jax.ShapeDtypeStruct((g_total, _N, _LANES), jnp.float32),
        ),
        mesh=mesh,
        compiler_params=pltpu.CompilerParams(needs_layout_passes=False),
        scratch_types=group_bufs + [pltpu.SemaphoreType.DMA],
    )
    return f(theta3, u3)


_U3_CACHE = {}


def _rotl32(x, r):
    return ((x << np.uint32(r)) | (x >> np.uint32(32 - r))).astype(np.uint32)


def _threefry2x32(k1, k2, x0, x1):
    ks = [np.uint32(k1), np.uint32(k2),
          np.uint32(np.uint32(k1) ^ np.uint32(k2) ^ np.uint32(0x1BD11BDA))]
    rots = [[13, 15, 26, 6], [17, 29, 16, 24]]
    x0 = (x0 + ks[0]).astype(np.uint32)
    x1 = (x1 + ks[1]).astype(np.uint32)
    for i in range(5):
        for r in rots[i % 2]:
            x0 = (x0 + x1).astype(np.uint32)
            x1 = _rotl32(x1, r)
            x1 = (x1 ^ x0).astype(np.uint32)
        x0 = (x0 + ks[(i + 1) % 3]).astype(np.uint32)
        x1 = (x1 + ks[(i + 2) % 3] + np.uint32(i + 1)).astype(np.uint32)
    return x0, x1


def _np_uniform_bits(seed, size):
    k1, k2 = np.uint32(seed >> 32), np.uint32(seed & 0xFFFFFFFF)
    if jax.config.jax_threefry_partitionable:
        cnt = np.arange(size, dtype=np.uint64)
        x0, x1 = _threefry2x32(k1, k2,
                               (cnt >> np.uint64(32)).astype(np.uint32),
                               (cnt & np.uint64(0xFFFFFFFF)).astype(np.uint32))
        bits = (x0 ^ x1).astype(np.uint32)
    else:
        cnt = np.arange(size, dtype=np.uint32)
        x0, x1 = _threefry2x32(k1, k2, cnt[:size // 2], cnt[size // 2:])
        bits = np.concatenate([x0, x1])
    fl = ((bits >> np.uint32(9)) | np.uint32(0x3F800000)).view(np.float32)
    return np.maximum(np.float32(0.0), fl - np.float32(1.0))


def _uniform_blocks(choices, rows):
    key = (choices, rows, bool(jax.config.jax_threefry_partitionable))
    if key not in _U3_CACHE:
        uni = _np_uniform_bits(42, choices * rows)
        groups = rows // _LANES
        u3 = uni.reshape(choices, groups, _LANES).transpose(1, 0, 2)
        _U3_CACHE[key] = np.ascontiguousarray(u3)
    return _U3_CACHE[key]


def kernel(scores):
    nnodes, choices, ensemble = scores.shape
    rows = nnodes * ensemble
    groups = rows // _LANES
    flat = jnp.transpose(scores, (0, 2, 1)).reshape(rows, choices)
    theta3 = flat.reshape(groups, _LANES, choices).transpose(0, 2, 1)
    u3 = jnp.asarray(_uniform_blocks(choices, rows))

    mask3, marg3 = _sc_sampler(theta3, u3)

    samples = mask3.transpose(0, 2, 1).reshape(rows, choices)
    marg = marg3.transpose(0, 2, 1).reshape(rows, choices)
    new_mask = samples.reshape(1, nnodes, ensemble, choices)
    new_mask = jnp.transpose(new_mask, (0, 1, 3, 2))
    new_marg = jnp.transpose(marg.reshape(nnodes, ensemble, choices),
                             (0, 2, 1))
    return new_mask, new_marg


# speedup vs baseline: 7.1234x; 1.2101x over previous
import functools

import jax
import jax.numpy as jnp
import numpy as np
from jax import lax
from jax.experimental import pallas as pl
from jax.experimental.pallas import tpu as pltpu
from jax.experimental.pallas import tpu_sc as plsc

_K = 8
_N = 64
_LANES = 16
_NUM_CORES = 2
_NUM_SUBCORES = 16
_NUM_WORKERS = _NUM_CORES * _NUM_SUBCORES
_BR = _K + 1
_PAIR = 2
_W = _PAIR * _LANES


def _sc_body(theta_hbm, u_hbm, mask_hbm, marg_hbm,
             th_v, u_v, w_v, btab, mask_v, marg_v, in_sem,
             *, blocks_per_worker):
    wid = lax.axis_index("s") * _NUM_CORES + lax.axis_index("c")
    lane = jnp.arange(_LANES, dtype=jnp.int32)
    lanes = [lane + 16 * t for t in range(_PAIR)]
    halves = [pl.ds(16 * t, 16) for t in range(_PAIR)]
    one = jnp.ones((_LANES,), jnp.float32)
    zero = jnp.zeros((_LANES,), jnp.float32)

    for h in halves:
        for j in range(_N + 1):
            btab[j * _BR, h] = one
        for r in range(1, _BR):
            btab[_N * _BR + r, h] = zero

    def do_block(i, carry):
        b = wid * blocks_per_worker + i
        c1 = pltpu.async_copy(theta_hbm.at[b], th_v, in_sem)
        c2 = pltpu.async_copy(u_hbm.at[b], u_v, in_sem)
        c1.wait()
        c2.wait()

        acc = [th_v[0, h] for h in halves]
        for j in range(1, _N):
            acc = [a + th_v[j, h] for a, h in zip(acc, halves)]
        mu = [a * jnp.float32(1.0 / _N) for a in acc]

        bs = [[one] + [zero] * _K for _ in range(_PAIR)]
        for j in range(_N - 1, -1, -1):
            wj = [jnp.exp(th_v[j, h] - m) for h, m in zip(halves, mu)]
            for t, h in enumerate(halves):
                w_v[j, h] = wj[t]
                for r in range(_K, 0, -1):
                    bs[t][r] = bs[t][r] + wj[t] * bs[t][r - 1]
                for r in range(1, _BR):
                    btab[j * _BR + r, h] = bs[t][r]

        inv_ek = [one / btab[_K, h] for h in halves]

        fs = [[one] + [zero] * (_K - 1) for _ in range(_PAIR)]
        rem = [jnp.full((_LANES,), _K, jnp.int32) for _ in range(_PAIR)]
        for j in range(_N):
            base1 = (j + 1) * _BR
            jv = jnp.full((_LANES,), j * _BR, jnp.int32)
            for t, h in enumerate(halves):
                wj = w_v[j, h]
                dot = fs[t][_K - 1] + fs[t][0] * btab[base1 + _K - 1, h]
                for r in range(1, _K - 1):
                    dot = dot + fs[t][r] * btab[base1 + _K - 1 - r, h]
                marg_v[j, h] = wj * dot * inv_ek[t]
                for r in range(_K - 1, 0, -1):
                    fs[t][r] = fs[t][r] + wj * fs[t][r - 1]

                idx = jv + rem[t]
                b_cur = plsc.load_gather(btab, [idx, lanes[t]])
                b_inc = plsc.load_gather(btab, [idx + _K, lanes[t]])
                p = wj * b_inc / jnp.maximum(b_cur, jnp.float32(1e-35))
                inc = (u_v[j, h] < p) & (rem[t] > 0)
                rem[t] = rem[t] - inc.astype(jnp.int32)
                mask_v[j, h] = inc.astype(jnp.float32)

        pltpu.sync_copy(mask_v, mask_hbm.at[b])
        pltpu.sync_copy(marg_v, marg_hbm.at[b])
        return carry

    lax.fori_loop(0, blocks_per_worker, do_block, 0)


@jax.jit
def _sc_sampler(theta3, u3):
    b_total = theta3.shape[0]
    blocks_per_worker = b_total // _NUM_WORKERS
    mesh = plsc.VectorSubcoreMesh(
        core_axis_name="c", subcore_axis_name="s",
        num_cores=_NUM_CORES, num_subcores=_NUM_SUBCORES)
    body = functools.partial(_sc_body, blocks_per_worker=blocks_per_worker)
    f = pl.kernel(
        body,
        out_type=(
            jax.ShapeDtypeStruct((b_total, _N, _W), jnp.float32),
            jax.ShapeDtypeStruct((b_total, _N, _W), jnp.float32),
        ),
        mesh=mesh,
        compiler_params=pltpu.CompilerParams(needs_layout_passes=False),
        scratch_types=[
            pltpu.VMEM((_N, _W), jnp.float32),
            pltpu.VMEM((_N, _W), jnp.float32),
            pltpu.VMEM((_N, _W), jnp.float32),
            pltpu.VMEM(((_N + 1) * _BR, _W), jnp.float32),
            pltpu.VMEM((_N, _W), jnp.float32),
            pltpu.VMEM((_N, _W), jnp.float32),
            pltpu.SemaphoreType.DMA,
        ],
    )
    return f(theta3, u3)


_U3_CACHE = {}


def _rotl32(x, r):
    return ((x << np.uint32(r)) | (x >> np.uint32(32 - r))).astype(np.uint32)


def _threefry2x32(k1, k2, x0, x1):
    ks = [np.uint32(k1), np.uint32(k2),
          np.uint32(np.uint32(k1) ^ np.uint32(k2) ^ np.uint32(0x1BD11BDA))]
    rots = [[13, 15, 26, 6], [17, 29, 16, 24]]
    x0 = (x0 + ks[0]).astype(np.uint32)
    x1 = (x1 + ks[1]).astype(np.uint32)
    for i in range(5):
        for r in rots[i % 2]:
            x0 = (x0 + x1).astype(np.uint32)
            x1 = _rotl32(x1, r)
            x1 = (x1 ^ x0).astype(np.uint32)
        x0 = (x0 + ks[(i + 1) % 3]).astype(np.uint32)
        x1 = (x1 + ks[(i + 2) % 3] + np.uint32(i + 1)).astype(np.uint32)
    return x0, x1


def _np_uniform_bits(seed, size):
    k1, k2 = np.uint32(seed >> 32), np.uint32(seed & 0xFFFFFFFF)
    if jax.config.jax_threefry_partitionable:
        cnt = np.arange(size, dtype=np.uint64)
        x0, x1 = _threefry2x32(k1, k2,
                               (cnt >> np.uint64(32)).astype(np.uint32),
                               (cnt & np.uint64(0xFFFFFFFF)).astype(np.uint32))
        bits = (x0 ^ x1).astype(np.uint32)
    else:
        cnt = np.arange(size, dtype=np.uint32)
        x0, x1 = _threefry2x32(k1, k2, cnt[:size // 2], cnt[size // 2:])
        bits = np.concatenate([x0, x1])
    fl = ((bits >> np.uint32(9)) | np.uint32(0x3F800000)).view(np.float32)
    return np.maximum(np.float32(0.0), fl - np.float32(1.0))


def _uniform_blocks(choices, rows):
    key = (choices, rows, bool(jax.config.jax_threefry_partitionable))
    if key not in _U3_CACHE:
        uni = _np_uniform_bits(42, choices * rows)
        blocks = rows // _W
        u3 = uni.reshape(choices, blocks, _W).transpose(1, 0, 2)
        _U3_CACHE[key] = np.ascontiguousarray(u3)
    return _U3_CACHE[key]


def kernel(scores):
    nnodes, choices, ensemble = scores.shape
    rows = nnodes * ensemble
    blocks = rows // _W
    flat = jnp.transpose(scores, (0, 2, 1)).reshape(rows, choices)
    theta3 = flat.reshape(blocks, _W, choices).transpose(0, 2, 1)
    u3 = jnp.asarray(_uniform_blocks(choices, rows))

    mask3, marg3 = _sc_sampler(theta3, u3)

    samples = mask3.transpose(0, 2, 1).reshape(rows, choices)
    marg = marg3.transpose(0, 2, 1).reshape(rows, choices)
    new_mask = samples.reshape(1, nnodes, ensemble, choices)
    new_mask = jnp.transpose(new_mask, (0, 1, 3, 2))
    new_marg = jnp.transpose(marg.reshape(nnodes, ensemble, choices),
                             (0, 2, 1))
    return new_mask, new_marg
